# baseline (speedup 1.0000x reference)
import functools
import jax
import jax.numpy as jnp
from jax import lax
from jax.experimental import pallas as pl
from jax.experimental.pallas import tpu as pltpu
from jax.experimental.pallas import tpu_sc as plsc

_B = 16384
_KS0 = 0
_KS1 = 42
_KS2 = _KS0 ^ _KS1 ^ 0x1BD11BDA
_ROTS = ((13, 15, 26, 6), (17, 29, 16, 24))

_NW = 32
_ROWS_PER_W = _B // _NW
_CHUNKS = _ROWS_PER_W // 16


def _sc_body(wb_hbm, out0_hbm, out1_hbm, wb_v, col0_v, col1_v):
    wid = lax.axis_index("s") * 2 + lax.axis_index("c")
    base_row = wid * _ROWS_PER_W

    pltpu.sync_copy(wb_hbm, wb_v)
    cp0_j0 = wb_v[4] + wb_v[0]
    cp0_j1 = wb_v[5] + wb_v[2]
    cp1_j0 = wb_v[4] + wb_v[1]
    cp1_j1 = wb_v[5] + wb_v[3]

    iota = lax.iota(jnp.int32, 16)
    ks = (jnp.uint32(_KS0), jnp.uint32(_KS1), jnp.uint32(_KS2))

    for c in range(_CHUNKS):
        rows_global = (iota + (base_row + c * 16)).astype(jnp.uint32)
        x0 = jnp.zeros((16,), jnp.uint32) + ks[0]
        x1 = rows_global + ks[1]
        for rnd in range(5):
            for rot in _ROTS[rnd % 2]:
                x0 = x0 + x1
                x1 = x0 ^ ((x1 << rot) | (x1 >> (32 - rot)))
            x0 = x0 + ks[(rnd + 1) % 3]
            x1 = x1 + ks[(rnd + 2) % 3] + jnp.uint32(rnd + 1)
        bits = x0 ^ x1
        sel = (bits >> 31) == 0
        col0_v[pl.ds(c * 16, 16)] = jnp.where(sel, cp1_j0, cp0_j0)
        col1_v[pl.ds(c * 16, 16)] = jnp.where(sel, cp1_j1, cp0_j1)

    pltpu.sync_copy(col0_v, out0_hbm.at[pl.ds(base_row, _ROWS_PER_W)])
    pltpu.sync_copy(col1_v, out1_hbm.at[pl.ds(base_row, _ROWS_PER_W)])


@functools.cache
def _build():
    mesh = plsc.VectorSubcoreMesh(core_axis_name="c", subcore_axis_name="s")
    return pl.kernel(
        _sc_body,
        mesh=mesh,
        out_type=[jax.ShapeDtypeStruct((_B,), jnp.float32),
                  jax.ShapeDtypeStruct((_B,), jnp.float32)],
        scratch_types=[
            pltpu.VMEM((8, 16), jnp.float32),
            pltpu.VMEM((_ROWS_PER_W,), jnp.float32),
            pltpu.VMEM((_ROWS_PER_W,), jnp.float32),
        ],
    )


def kernel(input_ids, attention_mask, W, b):
    wb = jnp.concatenate([W.reshape(-1).astype(jnp.float32),
                          b.astype(jnp.float32),
                          jnp.zeros((2,), jnp.float32)])
    wb = jnp.broadcast_to(wb[:, None], (8, 16))
    col0, col1 = _build()(wb)
    return (col0, col1)


# speedup vs baseline: 7.5442x; 7.5442x over previous
import jax
import jax.numpy as jnp
from jax.experimental import pallas as pl
from jax.experimental.pallas import tpu as pltpu

_B = 16384
_R = 128
_KS0 = 0
_KS1 = 42
_KS2 = _KS0 ^ _KS1 ^ 0x1BD11BDA
_ROTS = ((13, 15, 26, 6), (17, 29, 16, 24))


def _rng_select_kernel(w_ref, b_ref, col0_ref, col1_ref):
    r = jax.lax.broadcasted_iota(jnp.uint32, (_R, 128), 0)
    c = jax.lax.broadcasted_iota(jnp.uint32, (_R, 128), 1)
    i = r * jnp.uint32(128) + c

    ks = (jnp.uint32(_KS0), jnp.uint32(_KS1), jnp.uint32(_KS2))
    x0 = jnp.full((_R, 128), ks[0], dtype=jnp.uint32)
    x1 = i + ks[1]
    for rnd in range(5):
        for rot in _ROTS[rnd % 2]:
            x0 = x0 + x1
            x1 = x0 ^ ((x1 << rot) | (x1 >> (32 - rot)))
        x0 = x0 + ks[(rnd + 1) % 3]
        x1 = x1 + ks[(rnd + 2) % 3] + jnp.uint32(rnd + 1)
    bits = x0 ^ x1

    sel = (bits >> 31) == 0
    cp1_j0 = b_ref[0] + w_ref[0, 1]
    cp1_j1 = b_ref[1] + w_ref[1, 1]
    cp0_j0 = b_ref[0] + w_ref[0, 0]
    cp0_j1 = b_ref[1] + w_ref[1, 0]
    col0_ref[...] = jnp.where(sel, cp1_j0, cp0_j0).reshape(_B)
    col1_ref[...] = jnp.where(sel, cp1_j1, cp0_j1).reshape(_B)


def kernel(input_ids, attention_mask, W, b):
    col0, col1 = pl.pallas_call(
        _rng_select_kernel,
        out_shape=[jax.ShapeDtypeStruct((_B,), jnp.float32),
                   jax.ShapeDtypeStruct((_B,), jnp.float32)],
        in_specs=[pl.BlockSpec(memory_space=pltpu.SMEM),
                  pl.BlockSpec(memory_space=pltpu.SMEM)],
    )(W.astype(jnp.float32), b.astype(jnp.float32))
    return jnp.stack([col0, col1], axis=1)


# speedup vs baseline: 12.9914x; 1.7221x over previous
import jax
import jax.numpy as jnp
from jax.experimental import pallas as pl
from jax.experimental.pallas import tpu as pltpu

_B = 16384
_R = 256
_KS0 = 0
_KS1 = 42
_KS2 = _KS0 ^ _KS1 ^ 0x1BD11BDA
_ROTS = ((13, 15, 26, 6), (17, 29, 16, 24))


def _rng_select_kernel(w_ref, b_ref, out_ref):
    rr = jax.lax.broadcasted_iota(jnp.uint32, (_R, 128), 0)
    l = jax.lax.broadcasted_iota(jnp.uint32, (_R, 128), 1)
    i = (rr >> 1) * jnp.uint32(128) + l
    j = rr & jnp.uint32(1)

    ks = (jnp.uint32(_KS0), jnp.uint32(_KS1), jnp.uint32(_KS2))
    x0 = jnp.full((_R, 128), ks[0], dtype=jnp.uint32)
    x1 = i + ks[1]
    for rnd in range(5):
        for rot in _ROTS[rnd % 2]:
            x0 = x0 + x1
            x1 = x0 ^ ((x1 << rot) | (x1 >> (32 - rot)))
        x0 = x0 + ks[(rnd + 1) % 3]
        x1 = x1 + ks[(rnd + 2) % 3] + jnp.uint32(rnd + 1)
    bits = x0 ^ x1

    top = bits >> 31
    cp1_j0 = b_ref[0] + w_ref[0, 1]
    cp1_j1 = b_ref[1] + w_ref[1, 1]
    cp0_j0 = b_ref[0] + w_ref[0, 0]
    cp0_j1 = b_ref[1] + w_ref[1, 0]
    vp1 = jnp.where(j == 0, cp1_j0, cp1_j1)
    vp0 = jnp.where(j == 0, cp0_j0, cp0_j1)
    out_ref[...] = jnp.where(top == 0, vp1, vp0).reshape(_R * 128)


def kernel(input_ids, attention_mask, W, b):
    flat = pl.pallas_call(
        _rng_select_kernel,
        out_shape=jax.ShapeDtypeStruct((_R * 128,), jnp.float32),
        in_specs=[pl.BlockSpec(memory_space=pltpu.SMEM),
                  pl.BlockSpec(memory_space=pltpu.SMEM)],
    )(W.astype(jnp.float32), b.astype(jnp.float32))
    return flat.reshape(128, 2, 128).transpose(0, 2, 1).reshape(_B, 2)


# speedup vs baseline: 14.5957x; 1.1235x over previous
import jax
import jax.numpy as jnp
from jax.experimental import pallas as pl
from jax.experimental.pallas import tpu as pltpu

_B = 16384
_R = 128
_KS0 = 0
_KS1 = 42
_KS2 = _KS0 ^ _KS1 ^ 0x1BD11BDA
_ROTS = ((13, 15, 26, 6), (17, 29, 16, 24))


def _rng_select_kernel(w_ref, b_ref, out_ref):
    r = jax.lax.broadcasted_iota(jnp.uint32, (_R, 128), 0)
    l = jax.lax.broadcasted_iota(jnp.uint32, (_R, 128), 1)
    i = r * jnp.uint32(128) + l

    ks = (jnp.uint32(_KS0), jnp.uint32(_KS1), jnp.uint32(_KS2))
    x0 = jnp.full((_R, 128), ks[0], dtype=jnp.uint32)
    x1 = i + ks[1]
    for rnd in range(5):
        for rot in _ROTS[rnd % 2]:
            x0 = x0 + x1
            x1 = x0 ^ ((x1 << rot) | (x1 >> (32 - rot)))
        x0 = x0 + ks[(rnd + 1) % 3]
        x1 = x1 + ks[(rnd + 2) % 3] + jnp.uint32(rnd + 1)
    bits = x0 ^ x1

    sel = (bits >> 31) == 0
    cp1_j0 = b_ref[0] + w_ref[0, 1]
    cp1_j1 = b_ref[1] + w_ref[1, 1]
    cp0_j0 = b_ref[0] + w_ref[0, 0]
    cp0_j1 = b_ref[1] + w_ref[1, 0]
    c0 = jnp.where(sel, cp1_j0, cp0_j0)
    c1 = jnp.where(sel, cp1_j1, cp0_j1)
    out_ref[:, 0, :] = c0
    out_ref[:, 1, :] = c1


def kernel(input_ids, attention_mask, W, b):
    out3 = pl.pallas_call(
        _rng_select_kernel,
        out_shape=jax.ShapeDtypeStruct((128, 2, 128), jnp.float32),
        in_specs=[pl.BlockSpec(memory_space=pltpu.SMEM),
                  pl.BlockSpec(memory_space=pltpu.SMEM)],
    )(W.astype(jnp.float32), b.astype(jnp.float32))
    return out3.transpose(0, 2, 1).reshape(_B, 2)


# speedup vs baseline: 14.6739x; 1.0054x over previous
import jax
import jax.numpy as jnp
from jax.experimental import pallas as pl
from jax.experimental.pallas import tpu as pltpu

_B = 16384
_R = 128
_KS0 = 0
_KS1 = 42
_KS2 = _KS0 ^ _KS1 ^ 0x1BD11BDA
_ROTS = ((13, 15, 26, 6), (17, 29, 16, 24))


def _rng_select_kernel(w_ref, b_ref, out_ref):
    r = jax.lax.broadcasted_iota(jnp.uint32, (_R, 128), 0)
    l = jax.lax.broadcasted_iota(jnp.uint32, (_R, 128), 1)
    i = r * jnp.uint32(128) + l

    ks = (jnp.uint32(_KS0), jnp.uint32(_KS1), jnp.uint32(_KS2))
    x0 = jnp.full((_R, 128), ks[0], dtype=jnp.uint32)
    x1 = i + ks[1]
    for rnd in range(5):
        for rot in _ROTS[rnd % 2]:
            x0 = x0 + x1
            x1 = x0 ^ ((x1 << rot) | (x1 >> (32 - rot)))
        x0 = x0 + ks[(rnd + 1) % 3]
        x1 = x1 + ks[(rnd + 2) % 3] + jnp.uint32(rnd + 1)
    bits = x0 ^ x1

    sel = (bits >> 31) == 0
    cp1_j0 = b_ref[0] + w_ref[0, 1]
    cp1_j1 = b_ref[1] + w_ref[1, 1]
    cp0_j0 = b_ref[0] + w_ref[0, 0]
    cp0_j1 = b_ref[1] + w_ref[1, 0]
    out_ref[:, 0, :] = jnp.where(sel, cp1_j0, cp0_j0)
    out_ref[:, 1, :] = jnp.where(sel, cp1_j1, cp0_j1)


def kernel(input_ids, attention_mask, W, b):
    out3 = pl.pallas_call(
        _rng_select_kernel,
        out_shape=jax.ShapeDtypeStruct((_R, 2, 128), jnp.float32),
        in_specs=[pl.BlockSpec(memory_space=pltpu.SMEM),
                  pl.BlockSpec(memory_space=pltpu.SMEM)],
    )(W.astype(jnp.float32), b.astype(jnp.float32))
    return out3.transpose(0, 2, 1).reshape(_B, 2)
